# baseline (speedup 1.0000x reference)
import functools

import jax
import jax.numpy as jnp
from jax import lax
from jax.experimental import pallas as pl
from jax.experimental.pallas import tpu as pltpu
from jax.experimental.pallas import tpu_sc as plsc

N = 10000
E = 160000
D = 256
DH = 128
NS = 16
L = 16
NPAD = 10240
RT = NPAD // NS
EPT = 10880
CR = 128
NR = EPT // CR
C = 64
NCH = EPT // C
NB = 2


def _mm_body(x_ref, w_ref, h0_ref, h1_ref):
    h = jnp.dot(x_ref[...], w_ref[...], preferred_element_type=jnp.float32)
    h0_ref[...] = h[:, :DH]
    h1_ref[...] = h[:, DH:]


def _matmul(x_pad, W_gcn):
    blk = 512
    return pl.pallas_call(
        _mm_body,
        grid=(NPAD // blk,),
        in_specs=[
            pl.BlockSpec((blk, D), lambda i: (i, 0)),
            pl.BlockSpec((D, D), lambda i: (0, 0)),
        ],
        out_specs=[pl.BlockSpec((blk, DH), lambda i: (i, 0))] * 2,
        out_shape=[jax.ShapeDtypeStruct((NPAD, DH), jnp.float32)] * 2,
    )(x_pad, W_gcn)


def _sc_agg(pk3, ew3, h0, h1):
    mesh = plsc.VectorSubcoreMesh(core_axis_name="c", subcore_axis_name="s")

    @functools.partial(
        pl.kernel,
        out_type=[jax.ShapeDtypeStruct((NPAD, DH), jnp.float32)] * 2,
        mesh=mesh,
        compiler_params=pltpu.CompilerParams(needs_layout_passes=False),
        scratch_types=[
            pltpu.VMEM_SHARED((NPAD, DH), jnp.float32),
            pltpu.VMEM_SHARED((NPAD,), jnp.float32),
            pltpu.VMEM((NR, CR), jnp.int32),
            pltpu.VMEM((NR, CR), jnp.float32),
            pltpu.VMEM((RT,), jnp.float32),
            pltpu.VMEM((CR,), jnp.int32),
            pltpu.VMEM((NB, C), jnp.int32),
            pltpu.VMEM((NB, C), jnp.int32),
            pltpu.VMEM((NB, C), jnp.float32),
            pltpu.VMEM((NB, C), jnp.float32),
            pltpu.VMEM((C, DH), jnp.float32),
            pltpu.VMEM((C, DH), jnp.float32),
            pltpu.SemaphoreType.DMA,
            pltpu.SemaphoreType.DMA,
            pltpu.SemaphoreType.DMA,
            pltpu.SemaphoreType.DMA,
            pltpu.SemaphoreType.DMA,
            pltpu.SemaphoreType.DMA,
            pltpu.SemaphoreType.DMA,
            pltpu.SemaphoreType.DMA,
        ],
    )
    def sc_kernel(pk_hbm, ew_hbm, h0_hbm, h1_hbm, agg0_hbm, agg1_hbm,
                  acc, deg, pkv, ewv, wbuf, colb3, rowb, colb, drb, dcb,
                  gbuf0, gbuf1, hs0, hs1, ds0, ds1, es0, es1, ss0, ss1):
        c = lax.axis_index("c")
        s = lax.axis_index("s")
        rbase = s * RT
        m14 = jnp.full((L,), 0x3FFF, jnp.int32)

        pltpu.sync_copy(pk_hbm.at[s], pkv)
        pltpu.sync_copy(ew_hbm.at[s], ewv)

        @pl.loop(0, RT // L)
        def _(k):
            wbuf[pl.ds(k * L, L)] = jnp.zeros((L,), jnp.float32)
        pltpu.sync_copy(wbuf, deg.at[pl.ds(rbase, RT)])

        @pl.loop(0, C)
        def _(i):
            for jj in range(DH // L):
                gbuf0[i, pl.ds(jj * L, L)] = jnp.zeros((L,), jnp.float32)

        @pl.loop(0, RT // C)
        def _(t):
            pltpu.sync_copy(gbuf0, acc.at[pl.ds(rbase + t * C, C)])
        plsc.subcore_barrier()

        @pl.loop(0, NR)
        def _(j):
            for i in range(CR // L):
                sl = pl.ds(i * L, L)
                colb3[sl] = lax.shift_right_logical(pkv[j, sl], 14)
            pltpu.sync_copy(ewv.at[j], deg.at[colb3], add=True)
        plsc.subcore_barrier()

        pltpu.sync_copy(deg.at[pl.ds(rbase, RT)], wbuf)

        @pl.loop(0, RT // L)
        def _(k):
            d = wbuf[pl.ds(k * L, L)]
            i = lax.bitcast_convert_type(d, jnp.int32)
            i = jnp.full((L,), 0x5F3759DF, jnp.int32) - lax.shift_right_logical(i, 1)
            y = lax.bitcast_convert_type(i, jnp.float32)
            for _ in range(3):
                y = y * (1.5 - 0.5 * d * y * y)
            wbuf[pl.ds(k * L, L)] = y
        pltpu.sync_copy(wbuf, deg.at[pl.ds(rbase, RT)])
        plsc.subcore_barrier()

        def _pass(hk_hbm, agg_hbm):
            bufs = (gbuf0, gbuf1)
            hsem = (hs0, hs1)
            dsem = (ds0, ds1)
            esem = (es0, es1)
            ssem = (ss0, ss1)

            def issue(q, b):
                q2 = q // 2
                qb = q % 2
                for i in range(C // L):
                    p = pkv[q2, pl.ds(qb * C + i * L, L)]
                    rowb[b, pl.ds(i * L, L)] = p & m14
                    colb[b, pl.ds(i * L, L)] = lax.shift_right_logical(p, 14)
                pltpu.async_copy(hk_hbm.at[rowb.at[b]], bufs[b], hsem[b])
                pltpu.async_copy(deg.at[rowb.at[b]], drb.at[b], dsem[b])
                pltpu.async_copy(deg.at[colb.at[b]], dcb.at[b], esem[b])

            for b in range(NB):
                issue(b, b)

            @pl.loop(0, NCH, step=NB)
            def _(j0):
                j2 = j0 // 2
                for b in range(NB):
                    pltpu.make_async_copy(
                        hk_hbm.at[rowb.at[b]], bufs[b], hsem[b]).wait()
                    pltpu.make_async_copy(
                        deg.at[rowb.at[b]], drb.at[b], dsem[b]).wait()
                    pltpu.make_async_copy(
                        deg.at[colb.at[b]], dcb.at[b], esem[b]).wait()

                    @pl.loop(0, C // L)
                    def _(i16):
                        sl = pl.ds(i16 * L, L)
                        nvec = (drb[b, sl] * ewv[j2, pl.ds(b * C + i16 * L, L)]
                                * dcb[b, sl])
                        for k in range(L):
                            vv = jnp.full((L,), nvec[k], jnp.float32)
                            for jj in range(DH // L):
                                fsl = pl.ds(jj * L, L)
                                bufs[b][i16 * L + k, fsl] = (
                                    bufs[b][i16 * L + k, fsl] * vv)

                    pltpu.async_copy(bufs[b], acc.at[colb.at[b]], ssem[b],
                                     add=True)

                    @pl.when(j0 + b + NB < NCH)
                    def _():
                        pltpu.make_async_copy(
                            bufs[b], acc.at[colb.at[b]], ssem[b]).wait()
                        issue(j0 + b + NB, b)

            for b in range(NB):
                pltpu.make_async_copy(
                    bufs[b], acc.at[colb.at[b]], ssem[b]).wait()

            plsc.subcore_barrier()
            pltpu.sync_copy(acc.at[pl.ds(rbase, RT)],
                            agg_hbm.at[pl.ds(rbase, RT)])

        @pl.when(c == 0)
        def _():
            _pass(h0_hbm, agg0_hbm)

        @pl.when(c == 1)
        def _():
            _pass(h1_hbm, agg1_hbm)

    return sc_kernel(pk3, ew3, h0, h1)


def _head_body(a0, a1, b0, b1, w0, w1, bl, o_ref):
    acc = bl[...]
    for a, b, w in ((a0, b0, w0), (a1, b1, w1)):
        z = jnp.maximum(a[...] + b[...], 0.0)
        acc = acc + jnp.dot(z, w[...], preferred_element_type=jnp.float32)
    o_ref[...] = acc


def _head(aggs, b_gcn, W_lin, b_lin):
    blk = 400
    grid = N // blk
    bs = [b_gcn[i * DH:(i + 1) * DH].reshape(1, DH) for i in range(2)]
    ws = [W_lin[i * DH:(i + 1) * DH] for i in range(2)]
    bl = b_lin.reshape(1, 1)
    return pl.pallas_call(
        _head_body,
        grid=(grid,),
        in_specs=(
            [pl.BlockSpec((blk, DH), lambda i: (i, 0))] * 2
            + [pl.BlockSpec((1, DH), lambda i: (0, 0))] * 2
            + [pl.BlockSpec((DH, 1), lambda i: (0, 0))] * 2
            + [pl.BlockSpec((1, 1), lambda i: (0, 0))]
        ),
        out_specs=pl.BlockSpec((blk, 1), lambda i: (i, 0)),
        out_shape=jax.ShapeDtypeStruct((N, 1), jnp.float32),
    )(*aggs, *bs, *ws, bl)


def kernel(x, edge_index, edge_weight, W_gcn, b_gcn, W_lin, b_lin):
    f32 = jnp.float32
    row = edge_index[0].astype(jnp.int32)
    col = edge_index[1].astype(jnp.int32)
    ew = edge_weight.astype(f32)

    ept0 = E // NS
    spt = N // NS
    pad = EPT - ept0 - spt
    pk_t = ((col << 14) | row).reshape(NS, ept0)
    loop_n = jnp.arange(N, dtype=jnp.int32).reshape(NS, spt)
    pk_self = (loop_n << 14) | loop_n
    dummy = ((N + (jnp.arange(pad, dtype=jnp.int32) % (NPAD - N))) << 14)
    pk3 = jnp.concatenate(
        [pk_t, pk_self, jnp.broadcast_to(dummy, (NS, pad))], axis=1
    ).reshape(NS, NR, CR)
    ew3 = jnp.concatenate(
        [ew.reshape(NS, ept0), jnp.ones((NS, spt), f32),
         jnp.zeros((NS, pad), f32)], axis=1
    ).reshape(NS, NR, CR)

    x_pad = jnp.pad(x.astype(f32), ((0, NPAD - N), (0, 0)))
    h0, h1 = _matmul(x_pad, W_gcn.astype(f32))
    aggs = _sc_agg(pk3, ew3, h0, h1)
    return _head(aggs, b_gcn.astype(f32), W_lin.astype(f32), b_lin.astype(f32))
